# Initial kernel scaffold; baseline (speedup 1.0000x reference)
import jax
import jax.numpy as jnp
from jax.experimental import pallas as pl


def kernel(x, a, rho, theta_map):
    raise NotImplementedError("write your pallas kernel here")


# speedup vs baseline: 1.3159x; 1.3159x over previous
import functools

import jax
import jax.numpy as jnp
from jax import lax
from jax.experimental import pallas as pl
from jax.experimental.pallas import tpu as pltpu
from jax.experimental.pallas import tpu_sc as plsc

NB_STATES = 1000000
BATCH = 16384
LANES = 16
NUM_WORKERS = 32
B_PER_W = BATCH // NUM_WORKERS

LN2 = 0.6931471805599453
SQRT2 = 1.4142135623730951


def _log_f32(v):
    bits = lax.bitcast_convert_type(v, jnp.int32)
    e = (bits >> 23) - 127
    m = lax.bitcast_convert_type(
        (bits & 0x007FFFFF) | 0x3F800000, jnp.float32)
    big = m > SQRT2
    m = jnp.where(big, m * 0.5, m)
    e = e + jnp.where(big, 1, 0)
    z = m - 1.0
    s = z / (2.0 + z)
    s2 = s * s
    p = 2.0 * s * (1.0 + s2 * (1.0 / 3.0 + s2 * (0.2 + s2 * (1.0 / 7.0))))
    return e.astype(jnp.float32) * LN2 + p


def _sc_body(x_hbm, a_hbm, rho_hbm, theta_hbm, out_hbm,
             idx_v, a_v, tm_v, r_v, out_v, sem_t, sem_r):
    wid = lax.axis_index("s") * 2 + lax.axis_index("c")
    base = wid * B_PER_W
    pltpu.sync_copy(x_hbm.at[pl.ds(base, B_PER_W)], idx_v)
    ct = pltpu.async_copy(theta_hbm.at[idx_v], tm_v, sem_t)
    cr = pltpu.async_copy(rho_hbm.at[idx_v], r_v, sem_r)
    pltpu.sync_copy(a_hbm.at[pl.ds(base, B_PER_W)], a_v)
    ct.wait()
    cr.wait()
    for i in range(B_PER_W // LANES):
        sl = pl.ds(i * LANES, LANES)
        t = tm_v[sl]
        r = r_v[sl]
        av = a_v[sl]
        congestion = 2.0 * _log_f32(r + 1e-05)
        move = jnp.where(av != 0, jnp.float32(0.1), jnp.float32(0.0))
        out_v[sl] = 100.0 * t - congestion - move
    pltpu.sync_copy(out_v, out_hbm.at[pl.ds(base, B_PER_W)])


@jax.jit
def kernel(x, a, rho, theta_map):
    mesh = plsc.VectorSubcoreMesh(core_axis_name="c", subcore_axis_name="s")
    run = pl.kernel(
        _sc_body,
        mesh=mesh,
        out_type=jax.ShapeDtypeStruct((BATCH,), jnp.float32),
        scratch_types=[
            pltpu.VMEM((B_PER_W,), jnp.int32),
            pltpu.VMEM((B_PER_W,), jnp.int32),
            pltpu.VMEM((B_PER_W,), jnp.float32),
            pltpu.VMEM((B_PER_W,), jnp.float32),
            pltpu.VMEM((B_PER_W,), jnp.float32),
            pltpu.SemaphoreType.DMA,
            pltpu.SemaphoreType.DMA,
        ],
    )
    return run(x, a, rho, theta_map)
